# baseline (speedup 1.0000x reference)
import functools

import jax
import jax.numpy as jnp
from jax import lax
from jax.experimental import pallas as pl
from jax.experimental.pallas import tpu as pltpu
from jax.experimental.pallas import tpu_sc as plsc

_NC, _NS = 2, 16
_CHUNK = 800
_NORM_BLOCK = 2048


def _sc_gather(wv, idx4):
    num_idx = idx4.shape[0]
    dw = wv.shape[1]
    nw = _NC * _NS
    b_per_w = num_idx // nw
    mesh = plsc.VectorSubcoreMesh(core_axis_name="c", subcore_axis_name="s")

    @functools.partial(
        pl.kernel,
        mesh=mesh,
        out_type=jax.ShapeDtypeStruct((num_idx, dw), wv.dtype),
        scratch_types=[
            pltpu.VMEM((_CHUNK,), jnp.int32),
            pltpu.VMEM((_CHUNK, dw), jnp.float32),
            pltpu.SemaphoreType.DMA,
        ],
    )
    def gather_kernel(w_hbm, i_hbm, o_hbm, idx_v, rows_v, sem):
        wid = lax.axis_index("s") * _NC + lax.axis_index("c")
        base = wid * b_per_w

        @pl.loop(0, b_per_w, step=_CHUNK)
        def _(off):
            pltpu.sync_copy(i_hbm.at[pl.ds(base + off, _CHUNK)], idx_v)
            pltpu.async_copy(w_hbm.at[idx_v], rows_v, sem).wait()
            pltpu.sync_copy(rows_v, o_hbm.at[pl.ds(base + off, _CHUNK)])

    return gather_kernel(wv, idx4)


def _select_normalize(g, r4p, d):
    n, dw = g.shape
    nsub = dw // d
    rb = _NORM_BLOCK // 128

    def body(g_ref, r_ref, o_ref):
        gb = g_ref[...]
        qt = r_ref[...].T
        q = jnp.concatenate([qt[:, p:p + 1] for p in range(rb)], axis=0)
        acc = jnp.zeros((gb.shape[0], d), jnp.float32)
        for k in range(nsub):
            acc = jnp.where(q == float(k), gb[:, k * d:(k + 1) * d], acc)
        s = jnp.sum(acc * acc, axis=1, keepdims=True)
        o_ref[...] = acc / jnp.maximum(jnp.sqrt(s), 1e-12)

    return pl.pallas_call(
        body,
        grid=(n // _NORM_BLOCK,),
        in_specs=[
            pl.BlockSpec((_NORM_BLOCK, dw), lambda i: (i, 0)),
            pl.BlockSpec((rb, 128), lambda i: (i, 0)),
        ],
        out_specs=pl.BlockSpec((_NORM_BLOCK, d), lambda i: (i, 0)),
        out_shape=jax.ShapeDtypeStruct((n, d), jnp.float32),
    )(g, r4p)


def kernel(x, weight):
    b, h = x.shape
    n, d = weight.shape
    num_idx = b * h
    idxf = x.reshape(num_idx).astype(jnp.int32)
    wv = weight.reshape(n // 4, 4 * d)
    idx4 = idxf // 4
    r4p = (idxf % 4).astype(jnp.float32).reshape(num_idx // 128, 128)
    g = _sc_gather(wv, idx4)
    out = _select_normalize(g, r4p, d)
    return out.reshape(b, h, d)


# speedup vs baseline: 1.1381x; 1.1381x over previous
import functools

import jax
import jax.numpy as jnp
from jax import lax
from jax.experimental import pallas as pl
from jax.experimental.pallas import tpu as pltpu
from jax.experimental.pallas import tpu_sc as plsc

_NC, _NS = 2, 16
_CHUNK = 800
_BB = 32


def _sc_gather(wv, idx4):
    num_idx = idx4.shape[0]
    dw = wv.shape[1]
    nw = _NC * _NS
    b_per_w = num_idx // nw
    mesh = plsc.VectorSubcoreMesh(core_axis_name="c", subcore_axis_name="s")

    @functools.partial(
        pl.kernel,
        mesh=mesh,
        out_type=jax.ShapeDtypeStruct((num_idx, dw), wv.dtype),
        scratch_types=[
            pltpu.VMEM((_CHUNK,), jnp.int32),
            pltpu.VMEM((_CHUNK, dw), jnp.float32),
            pltpu.SemaphoreType.DMA,
        ],
    )
    def gather_kernel(w_hbm, i_hbm, o_hbm, idx_v, rows_v, sem):
        wid = lax.axis_index("s") * _NC + lax.axis_index("c")
        base = wid * b_per_w

        @pl.loop(0, b_per_w, step=_CHUNK)
        def _(off):
            pltpu.sync_copy(i_hbm.at[pl.ds(base + off, _CHUNK)], idx_v)
            pltpu.async_copy(w_hbm.at[idx_v], rows_v, sem).wait()
            pltpu.sync_copy(rows_v, o_hbm.at[pl.ds(base + off, _CHUNK)])

    return gather_kernel(wv, idx4)


def _select_normalize(g, r4, b, h, d):
    n, dw = g.shape
    nsub = dw // d
    rows = _BB * h

    def body(g_ref, r_ref, o_ref):
        gb = g_ref[...]
        qt = r_ref[...].T
        q = jnp.concatenate([qt[:, p:p + 1] for p in range(_BB)], axis=0)
        acc = jnp.zeros((rows, d), jnp.float32)
        for k in range(nsub):
            acc = jnp.where(q == float(k), gb[:, k * d:(k + 1) * d], acc)
        s = jnp.sum(acc * acc, axis=1, keepdims=True)
        acc = acc / jnp.maximum(jnp.sqrt(s), 1e-12)
        for p in range(_BB):
            o_ref[p, :, :] = acc[p * h:(p + 1) * h, :]

    return pl.pallas_call(
        body,
        grid=(b // _BB,),
        in_specs=[
            pl.BlockSpec((rows, dw), lambda i: (i, 0)),
            pl.BlockSpec((_BB, h), lambda i: (i, 0)),
        ],
        out_specs=pl.BlockSpec((_BB, h, d), lambda i: (i, 0, 0)),
        out_shape=jax.ShapeDtypeStruct((b, h, d), jnp.float32),
    )(g, r4)


def kernel(x, weight):
    b, h = x.shape
    n, d = weight.shape
    num_idx = b * h
    xi = x.astype(jnp.int32)
    wv = weight.reshape(n // 4, 4 * d)
    idx4 = (xi // 4).reshape(num_idx)
    r4 = (xi % 4).astype(jnp.float32)
    g = _sc_gather(wv, idx4)
    return _select_normalize(g, r4, b, h, d)


# speedup vs baseline: 1.2104x; 1.0635x over previous
import functools

import jax
import jax.numpy as jnp
from jax import lax
from jax.experimental import pallas as pl
from jax.experimental.pallas import tpu as pltpu
from jax.experimental.pallas import tpu_sc as plsc

_NC, _NS = 2, 16
_CHUNK = 800
_PACK_ROWS = 2000
_BB = 64


def _pack_table(weight):
    n, d = weight.shape
    dw = 4 * d
    np4 = n // 4
    nblk = np4 // _PACK_ROWS

    def body(w0, w1, w2, w3, o_ref):
        o_ref[:, 0 * d:1 * d] = w0[...]
        o_ref[:, 1 * d:2 * d] = w1[...]
        o_ref[:, 2 * d:3 * d] = w2[...]
        o_ref[:, 3 * d:4 * d] = w3[...]

    return pl.pallas_call(
        body,
        grid=(nblk,),
        in_specs=[
            pl.BlockSpec((_PACK_ROWS, d),
                         functools.partial(lambda k, i: (i + k * nblk, 0), k))
            for k in range(4)
        ],
        out_specs=pl.BlockSpec((_PACK_ROWS, dw), lambda i: (i, 0)),
        out_shape=jax.ShapeDtypeStruct((np4, dw), jnp.float32),
    )(weight, weight, weight, weight)


def _sc_gather(wv, idx4):
    num_idx = idx4.shape[0]
    dw = wv.shape[1]
    nw = _NC * _NS
    b_per_w = num_idx // nw
    mesh = plsc.VectorSubcoreMesh(core_axis_name="c", subcore_axis_name="s")

    @functools.partial(
        pl.kernel,
        mesh=mesh,
        out_type=jax.ShapeDtypeStruct((num_idx, dw), wv.dtype),
        scratch_types=[
            pltpu.VMEM((_CHUNK,), jnp.int32),
            pltpu.VMEM((_CHUNK, dw), jnp.float32),
            pltpu.SemaphoreType.DMA,
        ],
    )
    def gather_kernel(w_hbm, i_hbm, o_hbm, idx_v, rows_v, sem):
        wid = lax.axis_index("s") * _NC + lax.axis_index("c")
        base = wid * b_per_w

        @pl.loop(0, b_per_w, step=_CHUNK)
        def _(off):
            pltpu.sync_copy(i_hbm.at[pl.ds(base + off, _CHUNK)], idx_v)
            pltpu.async_copy(w_hbm.at[idx_v], rows_v, sem).wait()
            pltpu.sync_copy(rows_v, o_hbm.at[pl.ds(base + off, _CHUNK)])

    return gather_kernel(wv, idx4)


def _select_normalize(g, qw, b, h, d):
    n, dw = g.shape
    nsub = dw // d
    rows = _BB * h

    def body(g_ref, q_ref, o_ref):
        gb = g_ref[...]
        qb = q_ref[...]
        acc = jnp.zeros((rows, d), jnp.float32)
        for k in range(nsub):
            acc = jnp.where(qb == float(k), gb[:, k * d:(k + 1) * d], acc)
        s = jax.lax.dot_general(
            acc * acc, jnp.ones((d, 1), jnp.float32),
            (((1,), (0,)), ((), ())), preferred_element_type=jnp.float32)
        acc = acc / jnp.maximum(jnp.sqrt(s), 1e-12)
        for p in range(_BB):
            o_ref[p, :, :] = acc[p * h:(p + 1) * h, :]

    return pl.pallas_call(
        body,
        grid=(b // _BB,),
        in_specs=[
            pl.BlockSpec((rows, dw), lambda i: (i, 0)),
            pl.BlockSpec((rows, d), lambda i: (i, 0)),
        ],
        out_specs=pl.BlockSpec((_BB, h, d), lambda i: (i, 0, 0)),
        out_shape=jax.ShapeDtypeStruct((b, h, d), jnp.float32),
    )(g, qw)


def kernel(x, weight):
    b, h = x.shape
    n, d = weight.shape
    num_idx = b * h
    xi = x.astype(jnp.int32)
    np4 = n // 4
    idx4 = (xi % np4).reshape(num_idx)
    qw = jnp.broadcast_to(
        (xi // np4).astype(jnp.float32).reshape(num_idx, 1), (num_idx, d))
    wv = _pack_table(weight)
    g = _sc_gather(wv, idx4)
    return _select_normalize(g, qw, b, h, d)
